# baseline (speedup 1.0000x reference)
import functools

import jax
import jax.numpy as jnp
import numpy as np
from jax import lax
from jax.experimental import pallas as pl
from jax.experimental.pallas import tpu as pltpu
from jax.experimental.pallas import tpu_sc as plsc

_IMG_W = 224
_IMG_H = 224
_WPAD = 256
_PPIX = _IMG_H * _WPAD


def _mask_indices():
    bw = (_IMG_W - 3) // 2
    bh = (_IMG_H - 3) // 2
    lines_cnt = 2 * bw * bh + bw + bh
    mat = np.zeros((_IMG_H, _IMG_W), dtype=bool)
    for idx in range(lines_cnt):
        y1 = idx // (2 * bw + 1)
        r = idx % (2 * bw + 1)
        if r < bw:
            x1, x2, y2 = r, r + 1, y1
        else:
            x1, x2, y2 = r - bw, r - bw, y1 + 1
        px = x2 * 2 + (y2 - y1)
        py = y2 * 2 + (x2 - x1)
        mat[py, px] = True
    return np.flatnonzero(mat.reshape(-1)).astype(np.int32)


_GIDX = _mask_indices()
_OUT_DIM = int(_GIDX.shape[0])
_N_VEC = (_OUT_DIM + 15) // 16

_N_VECP = 1528
_SPLIT_VEC = 758
_SPLIT_PAIR = _SPLIT_VEC // 2
_N_PAIR = _N_VECP // 2
_H0_LO = _WPAD
_H0_LEN = 28128
_H1_LO = 110 * _WPAD
_H1_LEN = 28640

_SRC_PAD = (_GIDX // _IMG_W) * _WPAD + (_GIDX % _IMG_W)
_LIDX = np.zeros(_N_VECP * 16, dtype=np.int32)
_LIDX[:_OUT_DIM] = _SRC_PAD - np.where(
    np.arange(_OUT_DIM) < _SPLIT_VEC * 16, _H0_LO, _H1_LO
)
assert (_LIDX[: _SPLIT_VEC * 16] < _H0_LEN).all() and (_LIDX >= 0).all()
assert (_LIDX[_SPLIT_VEC * 16 :] < min(_H1_LEN, 1 << 15)).all()
_LPAIR = _LIDX.reshape(_N_PAIR, 2, 16)
_PIDX = (_LPAIR[:, 0, :] | (_LPAIR[:, 1, :] << 16)).astype(np.int32).reshape(-1)

_N_IMG = 256
_N_FULL_CHUNK = _OUT_DIM // 128
_TAIL = _OUT_DIM - _N_FULL_CHUNK * 128


def _make_sc_gather():
    info = plsc.get_sparse_core_info()
    nc, ns = info.num_cores, info.num_subcores
    nw = nc * ns
    imgs_per_w = _N_IMG // nw
    mesh = plsc.VectorSubcoreMesh(core_axis_name="c", subcore_axis_name="s")

    @functools.partial(
        pl.kernel,
        mesh=mesh,
        out_type=(
            jax.ShapeDtypeStruct((_N_IMG, _OUT_DIM), jnp.float32),
            jax.ShapeDtypeStruct((_N_IMG, 128), jnp.float32),
        ),
        scratch_types=[
            pltpu.VMEM((_N_PAIR * 16,), jnp.int32),
            pltpu.VMEM((_H0_LEN,), jnp.float32),
            pltpu.VMEM((_H1_LEN,), jnp.float32),
            pltpu.VMEM((_N_VECP * 16,), jnp.float32),
            pltpu.VMEM((_N_VECP * 16,), jnp.float32),
            pltpu.SemaphoreType.DMA,
            pltpu.SemaphoreType.DMA,
            pltpu.SemaphoreType.DMA,
        ],
        compiler_params=pltpu.CompilerParams(needs_layout_passes=False),
    )
    def sc_gather(
        x_hbm, idx_hbm, out_hbm, tail_hbm,
        idx_v, buf_a, buf_b, out_v0, out_v1, sem_a, sem_b, sem_out,
    ):
        wid = lax.axis_index("s") * nc + lax.axis_index("c")
        pltpu.sync_copy(idx_hbm, idx_v)
        row0 = wid * imgs_per_w

        def in_slices(i, lo, ln, buf):
            return x_hbm.at[pl.ds((row0 + i) * _PPIX + lo, ln)], buf

        def fire_in(i, lo, ln, buf, sem):
            src, dst = in_slices(i, lo, ln, buf)
            pltpu.async_copy(src, dst, sem)

        def wait_in(i, lo, ln, buf, sem):
            src, dst = in_slices(i, lo, ln, buf)
            pltpu.make_async_copy(src, dst, sem).wait()

        out_bufs = (out_v0, out_v1)

        def fire_out(ov, row):
            def fire_chunk(kt, carry):
                pltpu.async_copy(
                    ov.at[pl.ds(kt * 128, 128)],
                    out_hbm.at[row, pl.ds(kt * 128, 128)],
                    sem_out,
                )
                return carry

            lax.fori_loop(0, _N_FULL_CHUNK, fire_chunk, 0, unroll=8)
            pltpu.async_copy(
                ov.at[pl.ds(_N_FULL_CHUNK * 128, 128)],
                tail_hbm.at[row],
                sem_out,
            )

        def drain_out(ov, row):
            pltpu.make_async_copy(
                ov.at[pl.ds(0, (_N_FULL_CHUNK + 1) * 128)],
                x_hbm.at[pl.ds(0, (_N_FULL_CHUNK + 1) * 128)],
                sem_out,
            ).wait()

        fire_in(0, _H0_LO, _H0_LEN, buf_a, sem_a)
        fire_in(0, _H1_LO, _H1_LEN, buf_b, sem_b)
        for i in range(imgs_per_w):
            ov = out_bufs[i % 2]
            pv = out_bufs[(i - 1) % 2]
            prow = row0 + i - 1
            wait_in(i, _H0_LO, _H0_LEN, buf_a, sem_a)
            if i >= 2:
                drain_out(out_bufs[i % 2], row0 + i - 2)

            if i >= 1:
                @plsc.parallel_loop(0, _N_FULL_CHUNK, unroll=8)
                def gather_h0f(p, ov=ov, pv=pv, prow=prow):
                    pk = idx_v[pl.ds(p * 16, 16)]
                    lo = lax.bitwise_and(pk, 0xFFFF)
                    hi = lax.shift_right_logical(pk, 16)
                    ov[pl.ds(p * 32, 16)] = plsc.load_gather(buf_a, [lo])
                    ov[pl.ds(p * 32 + 16, 16)] = plsc.load_gather(buf_a, [hi])
                    pltpu.async_copy(
                        pv.at[pl.ds(p * 128, 128)],
                        out_hbm.at[prow, pl.ds(p * 128, 128)],
                        sem_out,
                    )

                pltpu.async_copy(
                    pv.at[pl.ds(_N_FULL_CHUNK * 128, 128)],
                    tail_hbm.at[prow],
                    sem_out,
                )
            else:

                @plsc.parallel_loop(0, _N_FULL_CHUNK, unroll=8)
                def gather_h0a(p, ov=ov):
                    pk = idx_v[pl.ds(p * 16, 16)]
                    lo = lax.bitwise_and(pk, 0xFFFF)
                    hi = lax.shift_right_logical(pk, 16)
                    ov[pl.ds(p * 32, 16)] = plsc.load_gather(buf_a, [lo])
                    ov[pl.ds(p * 32 + 16, 16)] = plsc.load_gather(buf_a, [hi])

            @plsc.parallel_loop(_N_FULL_CHUNK, _SPLIT_PAIR, unroll=8)
            def gather_h0b(p, ov=ov):
                pk = idx_v[pl.ds(p * 16, 16)]
                lo = lax.bitwise_and(pk, 0xFFFF)
                hi = lax.shift_right_logical(pk, 16)
                ov[pl.ds(p * 32, 16)] = plsc.load_gather(buf_a, [lo])
                ov[pl.ds(p * 32 + 16, 16)] = plsc.load_gather(buf_a, [hi])

            if i + 1 < imgs_per_w:
                fire_in(i + 1, _H0_LO, _H0_LEN, buf_a, sem_a)
            wait_in(i, _H1_LO, _H1_LEN, buf_b, sem_b)

            @plsc.parallel_loop(_SPLIT_PAIR, _N_PAIR, unroll=8)
            def gather_h1(p, ov=ov):
                pk = idx_v[pl.ds(p * 16, 16)]
                lo = lax.bitwise_and(pk, 0xFFFF)
                hi = lax.shift_right_logical(pk, 16)
                ov[pl.ds(p * 32, 16)] = plsc.load_gather(buf_b, [lo])
                ov[pl.ds(p * 32 + 16, 16)] = plsc.load_gather(buf_b, [hi])

            if i + 1 < imgs_per_w:
                fire_in(i + 1, _H1_LO, _H1_LEN, buf_b, sem_b)
        fire_out(out_bufs[(imgs_per_w - 1) % 2], row0 + imgs_per_w - 1)
        drain_out(out_bufs[imgs_per_w % 2], row0 + imgs_per_w - 2)
        drain_out(out_bufs[(imgs_per_w - 1) % 2], row0 + imgs_per_w - 1)

    return sc_gather


_SC_GATHER = _make_sc_gather()


def kernel(x):
    B, T, H, W, _ = x.shape
    xp = jnp.pad(x, ((0, 0), (0, 0), (0, 0), (0, _WPAD - W), (0, 0)))
    flat = xp.reshape(B * T * H * _WPAD)
    idx = jnp.asarray(_PIDX)
    out, tail = _SC_GATHER(flat, idx)
    out = jax.lax.dynamic_update_slice(
        out, tail[:, :_TAIL], (0, _N_FULL_CHUNK * 128)
    )
    return out.reshape(B, T, _OUT_DIM)


# speedup vs baseline: 1.0339x; 1.0339x over previous
import functools

import jax
import jax.numpy as jnp
import numpy as np
from jax import lax
from jax.experimental import pallas as pl
from jax.experimental.pallas import tpu as pltpu
from jax.experimental.pallas import tpu_sc as plsc

_IMG_W = 224
_IMG_H = 224
_WPAD = 256
_PPIX = _IMG_H * _WPAD


def _mask_indices():
    bw = (_IMG_W - 3) // 2
    bh = (_IMG_H - 3) // 2
    lines_cnt = 2 * bw * bh + bw + bh
    mat = np.zeros((_IMG_H, _IMG_W), dtype=bool)
    for idx in range(lines_cnt):
        y1 = idx // (2 * bw + 1)
        r = idx % (2 * bw + 1)
        if r < bw:
            x1, x2, y2 = r, r + 1, y1
        else:
            x1, x2, y2 = r - bw, r - bw, y1 + 1
        px = x2 * 2 + (y2 - y1)
        py = y2 * 2 + (x2 - x1)
        mat[py, px] = True
    return np.flatnonzero(mat.reshape(-1)).astype(np.int32)


_GIDX = _mask_indices()
_OUT_DIM = int(_GIDX.shape[0])
_N_VEC = (_OUT_DIM + 15) // 16

_N_VECP = 1528
_SPLIT_VEC = 758
_SPLIT_PAIR = _SPLIT_VEC // 2
_N_PAIR = _N_VECP // 2
_H0_LO = _WPAD
_H0_LEN = 28128
_H1_LO = 110 * _WPAD
_H1_LEN = 28640

_SRC_PAD = (_GIDX // _IMG_W) * _WPAD + (_GIDX % _IMG_W)
_LIDX = np.zeros(_N_VECP * 16, dtype=np.int32)
_LIDX[:_OUT_DIM] = _SRC_PAD - np.where(
    np.arange(_OUT_DIM) < _SPLIT_VEC * 16, _H0_LO, _H1_LO
)
assert (_LIDX[: _SPLIT_VEC * 16] < _H0_LEN).all() and (_LIDX >= 0).all()
assert (_LIDX[_SPLIT_VEC * 16 :] < min(_H1_LEN, 1 << 15)).all()
_LPAIR = _LIDX.reshape(_N_PAIR, 2, 16)
_PIDX = (_LPAIR[:, 0, :] | (_LPAIR[:, 1, :] << 16)).astype(np.int32).reshape(-1)

_N_IMG = 256
_N_FULL_CHUNK = _OUT_DIM // 128
_TAIL = _OUT_DIM - _N_FULL_CHUNK * 128


def _make_sc_gather():
    info = plsc.get_sparse_core_info()
    nc, ns = info.num_cores, info.num_subcores
    nw = nc * ns
    imgs_per_w = _N_IMG // nw
    mesh = plsc.VectorSubcoreMesh(core_axis_name="c", subcore_axis_name="s")

    @functools.partial(
        pl.kernel,
        mesh=mesh,
        out_type=(
            jax.ShapeDtypeStruct((_N_IMG, _OUT_DIM), jnp.float32),
            jax.ShapeDtypeStruct((_N_IMG, 128), jnp.float32),
        ),
        scratch_types=[
            pltpu.VMEM((_N_PAIR * 16,), jnp.int32),
            pltpu.VMEM((_H0_LEN,), jnp.float32),
            pltpu.VMEM((_H1_LEN,), jnp.float32),
            pltpu.VMEM((_N_VECP * 16,), jnp.float32),
            pltpu.VMEM((_N_VECP * 16,), jnp.float32),
            pltpu.SemaphoreType.DMA,
            pltpu.SemaphoreType.DMA,
            pltpu.SemaphoreType.DMA,
        ],
        compiler_params=pltpu.CompilerParams(needs_layout_passes=False),
    )
    def sc_gather(
        x_hbm, idx_hbm, out_hbm, tail_hbm,
        idx_v, buf_a, buf_b, out_v0, out_v1, sem_a, sem_b, sem_out,
    ):
        wid = lax.axis_index("s") * nc + lax.axis_index("c")
        pltpu.sync_copy(idx_hbm, idx_v)
        row0 = wid * imgs_per_w

        def in_slices(i, lo, ln, buf):
            return x_hbm.at[pl.ds((row0 + i) * _PPIX + lo, ln)], buf

        def fire_in(i, lo, ln, buf, sem):
            src, dst = in_slices(i, lo, ln, buf)
            pltpu.async_copy(src, dst, sem)

        def wait_in(i, lo, ln, buf, sem):
            src, dst = in_slices(i, lo, ln, buf)
            pltpu.make_async_copy(src, dst, sem).wait()

        out_bufs = (out_v0, out_v1)

        def fire_out(ov, row):
            def fire_chunk(kt, carry):
                pltpu.async_copy(
                    ov.at[pl.ds(kt * 128, 128)],
                    out_hbm.at[row, pl.ds(kt * 128, 128)],
                    sem_out,
                )
                return carry

            lax.fori_loop(0, _N_FULL_CHUNK, fire_chunk, 0, unroll=8)
            pltpu.async_copy(
                ov.at[pl.ds(_N_FULL_CHUNK * 128, 128)],
                tail_hbm.at[row],
                sem_out,
            )

        def drain_out(ov, row):
            pltpu.make_async_copy(
                ov.at[pl.ds(0, (_N_FULL_CHUNK + 1) * 128)],
                x_hbm.at[pl.ds(0, (_N_FULL_CHUNK + 1) * 128)],
                sem_out,
            ).wait()

        fire_in(0, _H0_LO, _H0_LEN, buf_a, sem_a)
        fire_in(0, _H1_LO, _H1_LEN, buf_b, sem_b)
        for i in range(imgs_per_w):
            ov = out_bufs[i % 2]
            wait_in(i, _H0_LO, _H0_LEN, buf_a, sem_a)
            if i >= 2:
                drain_out(out_bufs[i % 2], row0 + i - 2)

            @plsc.parallel_loop(0, _SPLIT_PAIR, unroll=8)
            def gather_h0(p, ov=ov):
                pk = idx_v[pl.ds(p * 16, 16)]
                lo = lax.bitwise_and(pk, 0xFFFF)
                hi = lax.shift_right_logical(pk, 16)
                ov[pl.ds(p * 32, 16)] = plsc.load_gather(buf_a, [lo])
                ov[pl.ds(p * 32 + 16, 16)] = plsc.load_gather(buf_a, [hi])

            if i + 1 < imgs_per_w:
                fire_in(i + 1, _H0_LO, _H0_LEN, buf_a, sem_a)
            wait_in(i, _H1_LO, _H1_LEN, buf_b, sem_b)

            @plsc.parallel_loop(_SPLIT_PAIR, _N_PAIR, unroll=8)
            def gather_h1(p, ov=ov):
                pk = idx_v[pl.ds(p * 16, 16)]
                lo = lax.bitwise_and(pk, 0xFFFF)
                hi = lax.shift_right_logical(pk, 16)
                ov[pl.ds(p * 32, 16)] = plsc.load_gather(buf_b, [lo])
                ov[pl.ds(p * 32 + 16, 16)] = plsc.load_gather(buf_b, [hi])

            if i + 1 < imgs_per_w:
                fire_in(i + 1, _H1_LO, _H1_LEN, buf_b, sem_b)
            fire_out(ov, row0 + i)
        drain_out(out_bufs[imgs_per_w % 2], row0 + imgs_per_w - 2)
        drain_out(out_bufs[(imgs_per_w - 1) % 2], row0 + imgs_per_w - 1)

    return sc_gather


_SC_GATHER = _make_sc_gather()


def kernel(x):
    B, T, H, W, _ = x.shape
    xp = jnp.pad(x, ((0, 0), (0, 0), (0, 0), (0, _WPAD - W), (0, 0)))
    flat = xp.reshape(B * T * H * _WPAD)
    idx = jnp.asarray(_PIDX)
    out, tail = _SC_GATHER(flat, idx)
    out = jax.lax.dynamic_update_slice(
        out, tail[:, :_TAIL], (0, _N_FULL_CHUNK * 128)
    )
    return out.reshape(B, T, _OUT_DIM)
